# baseline (speedup 1.0000x reference)
import functools

import jax
import jax.numpy as jnp
from jax import lax
from jax.experimental import pallas as pl
from jax.experimental.pallas import tpu as pltpu
from jax.experimental.pallas import tpu_sc as plsc

_NBUF = 16
_L = 16


@functools.lru_cache(maxsize=None)
def _make_lookup(vocab: int, embed: int, batch: int):
  info = plsc.get_sparse_core_info()
  nc, ns = info.num_cores, info.num_subcores
  jh = embed // nc
  bpw = batch // ns
  assert bpw % _NBUF == 0
  mesh = plsc.VectorSubcoreMesh(core_axis_name="c", subcore_axis_name="s")

  @functools.partial(
      pl.kernel,
      mesh=mesh,
      out_type=jax.ShapeDtypeStruct((embed, batch), jnp.float32),
      scratch_types=[
          pltpu.VMEM((bpw,), jnp.int32),
          pltpu.VMEM((bpw * jh,), jnp.float32),
          pltpu.VMEM((jh, bpw), jnp.float32),
          pltpu.SemaphoreType.DMA,
      ]
      + [pltpu.VMEM((jh, 128), jnp.float32) for _ in range(_NBUF)]
      + [pltpu.SemaphoreType.DMA for _ in range(_NBUF)],
      compiler_params=pltpu.CompilerParams(needs_layout_passes=False),
  )
  def lookup(emb_hbm, idx_hbm, out_hbm, idx_s, rows_v, t_v, sem_i, *ring):
    slabs = ring[:_NBUF]
    sems = ring[_NBUF:]
    iota = lax.iota(jnp.int32, _L)
    s = lax.axis_index("s")
    jb = pl.multiple_of(lax.axis_index("c") * jh, 8)
    base = s * bpw
    pltpu.async_copy(idx_hbm.at[pl.ds(base, bpw)], idx_s, sem_i).wait()

    def fire(r, ss):
      cb = pl.multiple_of((r // 128) * 128, 128)
      return pltpu.async_copy(
          emb_hbm.at[pl.ds(jb, jh), pl.ds(cb, 128)], slabs[ss], sems[ss]
      )

    def extract(r, i, ss):
      lane = jnp.full((_L,), r & 127, jnp.int32)
      col = plsc.load_gather(slabs[ss], [iota, lane])
      rows_v[pl.ds(i * jh, _L)] = col

    rv0 = idx_s[pl.ds(0, _NBUF)]
    for ss in range(_NBUF):
      fire(rv0[ss], ss)

    def step(g, carry):
      i = g * _NBUF
      rv = idx_s[pl.ds(i, _NBUF)]
      for ss in range(_NBUF):
        pltpu.make_async_copy(
            emb_hbm.at[pl.ds(0, jh), pl.ds(0, 128)], slabs[ss], sems[ss]
        ).wait()
        extract(rv[ss], i + ss, ss)

        @pl.when(g + 1 < bpw // _NBUF)
        def _():
          rvn = idx_s[pl.ds(i + _NBUF, _NBUF)]
          fire(rvn[ss], ss)

      return carry

    lax.fori_loop(0, bpw // _NBUF, step, 0)

    def tstep(g, carry):
      b16 = (g * _L + iota) * jh
      for j in range(jh):
        t_v[j, pl.ds(g * _L, _L)] = plsc.load_gather(rows_v, [b16 + j])
      return carry

    lax.fori_loop(0, bpw // _L, tstep, 0)
    pltpu.sync_copy(t_v, out_hbm.at[pl.ds(jb, jh), pl.ds(base, bpw)])

  return lookup


def kernel(inputs, embeddings):
  batch, = inputs.shape
  vocab, embed = embeddings.shape
  idx = inputs.astype(jnp.int32)
  out_t = _make_lookup(vocab, embed, batch)(embeddings.T, idx)
  return out_t.T


# speedup vs baseline: 1.4866x; 1.4866x over previous
import functools

import jax
import jax.numpy as jnp
from jax import lax
from jax.experimental import pallas as pl
from jax.experimental.pallas import tpu as pltpu
from jax.experimental.pallas import tpu_sc as plsc

_W = 16
_L = 16


@functools.lru_cache(maxsize=None)
def _make_lookup(vocab: int, embed: int, batch: int):
  info = plsc.get_sparse_core_info()
  nc, ns = info.num_cores, info.num_subcores
  jh = embed // nc
  bpw = batch // ns
  waves = bpw // _W
  assert waves % 2 == 0
  mesh = plsc.VectorSubcoreMesh(core_axis_name="c", subcore_axis_name="s")

  @functools.partial(
      pl.kernel,
      mesh=mesh,
      out_type=jax.ShapeDtypeStruct((embed, batch), jnp.float32),
      scratch_types=[
          pltpu.VMEM((bpw,), jnp.int32),
          pltpu.VMEM((jh, bpw), jnp.float32),
          pltpu.SemaphoreType.DMA,
          pltpu.SemaphoreType.DMA,
      ]
      + [pltpu.VMEM((jh, 128), jnp.float32) for _ in range(2 * _W)],
      compiler_params=pltpu.CompilerParams(needs_layout_passes=False),
  )
  def lookup(emb_hbm, idx_hbm, out_hbm, idx_s, t_v, sem_i, sem, *slabs):
    iota = lax.iota(jnp.int32, _L)
    s = lax.axis_index("s")
    jb = pl.multiple_of(lax.axis_index("c") * jh, 8)
    base = s * bpw
    pltpu.async_copy(idx_hbm.at[pl.ds(base, bpw)], idx_s, sem_i).wait()

    def fire_wave(g, bank):
      rv = idx_s[pl.ds(g * _W, _W)]
      cbv = (rv // 128) * 128
      for ss in range(_W):
        cb = pl.multiple_of(cbv[ss], 128)
        pltpu.async_copy(
            emb_hbm.at[pl.ds(jb, jh), pl.ds(cb, 128)],
            slabs[bank * _W + ss], sem,
        )

    def drain_wave():
      cp = pltpu.make_async_copy(
          emb_hbm.at[pl.ds(0, jh), pl.ds(0, 128)], slabs[0], sem
      )
      for _ in range(_W):
        cp.wait()

    def extract_wave(g, bank):
      rv = idx_s[pl.ds(g * _W, _W)]
      lanev = rv & 127
      for ss in range(_W):
        lane = jnp.full((_L,), lanev[ss], jnp.int32)
        col = plsc.load_gather(slabs[bank * _W + ss], [iota, lane])
        bcol = jnp.full((_L,), g * _W + ss, jnp.int32)
        plsc.store_scatter(t_v, [iota, bcol], col)

    fire_wave(0, 0)

    def step(g2, carry):
      g = g2 * 2
      for half in range(2):

        @pl.when(g + half + 1 < waves)
        def _():
          fire_wave(g + half + 1, 1 - half)

        drain_wave()
        extract_wave(g + half, half)
      return carry

    lax.fori_loop(0, waves // 2, step, 0)
    pltpu.sync_copy(t_v, out_hbm.at[pl.ds(jb, jh), pl.ds(base, bpw)])

  return lookup


def kernel(inputs, embeddings):
  batch, = inputs.shape
  vocab, embed = embeddings.shape
  idx = inputs.astype(jnp.int32)
  out_t = _make_lookup(vocab, embed, batch)(embeddings.T, idx)
  return out_t.T
